# baseline (speedup 1.0000x reference)
import functools

import jax
import jax.numpy as jnp
from jax import lax
from jax.experimental import pallas as pl
from jax.experimental.pallas import tpu as pltpu
from jax.experimental.pallas import tpu_sc as plsc

BATCH = 4096
SEQ = 200
EMBED = 64
PADW = 128
NC, NS, LANES = 2, 16, 16
NW = NC * NS
ROWS_PER_W = BATCH // NW
HALF = SEQ // 2
UNITS_PER_W = 2 * ROWS_PER_W
IDX_BLK = 64


def _body(idx_hbm, tok_hbm, pos_hbm, out_hbm, idx_v, pos_v, gbufs, sbufs,
          gsems, ssems):
    wid = lax.axis_index("s") * NC + lax.axis_index("c")

    pltpu.sync_copy(idx_hbm.at[pl.ds(wid * UNITS_PER_W, 2 * IDX_BLK)], idx_v)
    pltpu.sync_copy(pos_hbm, pos_v)

    def fire_gather(gb, u):
        pltpu.async_copy(
            tok_hbm.at[idx_v.at[u % (2 * IDX_BLK), pl.ds(0, HALF)]],
            gbufs[gb], gsems[gb])

    def wait_gather(gb):
        pltpu.make_async_copy(tok_hbm.at[idx_v.at[0, pl.ds(0, HALF)]],
                              gbufs[gb], gsems[gb]).wait()

    def fire_store(sb, r):
        pltpu.async_copy(sbufs[sb], out_hbm.at[wid * ROWS_PER_W + r],
                         ssems[sb])

    def wait_store(sb):
        pltpu.make_async_copy(sbufs[sb], out_hbm.at[0], ssems[sb]).wait()

    def add_pos(gb, sb, h):
        gbuf = gbufs[gb]
        sbuf = sbufs[sb]

        @plsc.parallel_loop(0, HALF, 1, unroll=4)
        def _(s):
            for d in range(EMBED // LANES):
                sl = pl.ds(d * LANES, LANES)
                sbuf[h * HALF + s, sl] = gbuf[s, sl] + pos_v[h * HALF + s, sl]

    fire_gather(0, 0)

    def row(r, _):
        u = 2 * r
        sb_dyn = r % 2

        rows_per_blk = IDX_BLK // 2
        blk = r // rows_per_blk

        @pl.when(jnp.logical_and(r % rows_per_blk == 1,
                                 r < ROWS_PER_W - rows_per_blk + 1))
        def _():
            pltpu.sync_copy(
                idx_hbm.at[pl.ds(wid * UNITS_PER_W + (blk + 1) * IDX_BLK,
                                 IDX_BLK)],
                idx_v.at[pl.ds(((blk + 1) % 2) * IDX_BLK, IDX_BLK)])

        for sb in range(2):

            @pl.when(sb_dyn == sb)
            def _():
                fire_gather(1, u + 1)
                wait_gather(0)

                @pl.when(r >= 2)
                def _():
                    wait_store(sb)

                add_pos(0, sb, 0)

                @pl.when(u + 2 < UNITS_PER_W)
                def _():
                    fire_gather(0, u + 2)

                wait_gather(1)
                add_pos(1, sb, 1)
                fire_store(sb, r)

        return ()

    lax.fori_loop(0, ROWS_PER_W, row, ())
    wait_store(0)
    wait_store(1)


@functools.partial(
    pl.kernel,
    out_type=jax.ShapeDtypeStruct((BATCH, SEQ, EMBED), jnp.float32),
    mesh=plsc.VectorSubcoreMesh(core_axis_name="c", subcore_axis_name="s",
                                num_cores=NC, num_subcores=NS),
    scratch_types=[
        pltpu.VMEM((2 * IDX_BLK, PADW), jnp.int32),
        pltpu.VMEM((SEQ, PADW), jnp.float32),
    ] + [pltpu.VMEM((HALF, PADW), jnp.float32)] * 2
      + [pltpu.VMEM((SEQ, EMBED), jnp.float32)] * 2
      + [pltpu.SemaphoreType.DMA] * 4,
)
def _embed_kernel(idx_hbm, tok_hbm, pos_hbm, out_hbm, idx_v, pos_v, *rest):
    gbufs = rest[0:2]
    sbufs = rest[2:4]
    gsems = rest[4:6]
    ssems = rest[6:8]
    _body(idx_hbm, tok_hbm, pos_hbm, out_hbm, idx_v, pos_v, gbufs, sbufs,
          gsems, ssems)


def kernel(inputs, token_table, pos_table):
    idx = inputs.astype(jnp.int32).reshape(BATCH * SEQ // HALF, HALF)
    idx = jnp.pad(idx, ((0, 0), (0, PADW - HALF)))
    tok = jnp.pad(token_table, ((0, 0), (0, PADW - EMBED)))
    pos = jnp.pad(pos_table, ((0, 0), (0, PADW - EMBED)))
    return _embed_kernel(idx, tok, pos)
